# baseline (speedup 1.0000x reference)
import functools

import jax
import jax.numpy as jnp
from jax import lax
from jax.experimental import pallas as pl
from jax.experimental.pallas import tpu as pltpu
from jax.experimental.pallas import tpu_sc as plsc

N = 10000
E = 320000
D = 128
NC = 2
NS = 16
NW = NC * NS
NP = 10240
RPT_DEG = NP // NS
EPT2 = E // NS
K = 80
EPT = E // NW
CH = EPT // K
ACC = N

f32 = jnp.float32

_mesh = plsc.VectorSubcoreMesh(
    core_axis_name="c", subcore_axis_name="s", num_cores=NC, num_subcores=NS)
_sc_params = pltpu.CompilerParams(needs_layout_passes=False)


@functools.partial(
    pl.kernel,
    out_type=jax.ShapeDtypeStruct((2, NP), f32),
    mesh=_mesh,
    scratch_types=[
        pltpu.VMEM((EPT2,), jnp.int32),
        pltpu.VMEM((NP,), f32),
        pltpu.VMEM((RPT_DEG,), f32),
        pltpu.VMEM((RPT_DEG,), f32),
        pltpu.VMEM_SHARED((NS, NP), f32),
    ],
    compiler_params=_sc_params,
)
def _deg_kernel(idx_hbm, deg_out, idx_v, deg_v, acc_v, tmp_v, deg_sh):
    c = lax.axis_index("c")
    s = lax.axis_index("s")
    row = c * NS + s
    pltpu.sync_copy(idx_hbm.at[row], idx_v)

    zero16 = jnp.zeros((16,), f32)
    ones16 = jnp.ones((16,), f32)

    def zbody(i, carry):
        deg_v[pl.ds(i * 16, 16)] = zero16
        return carry
    lax.fori_loop(0, NP // 16, zbody, None)

    def ebody(e, carry):
        idx = idx_v[pl.ds(e * 16, 16)]
        plsc.addupdate_scatter(deg_v, [idx], ones16)
        return carry
    lax.fori_loop(0, EPT2 // 16, ebody, None)

    pltpu.sync_copy(deg_v, deg_sh.at[s])
    plsc.subcore_barrier()

    base = s * RPT_DEG
    pltpu.sync_copy(deg_sh.at[0, pl.ds(base, RPT_DEG)], acc_v)
    for p in range(1, NS):
        pltpu.sync_copy(deg_sh.at[p, pl.ds(base, RPT_DEG)], tmp_v)

        def abody(i, carry):
            sl = pl.ds(i * 16, 16)
            acc_v[sl] = acc_v[sl] + tmp_v[sl]
            return carry
        lax.fori_loop(0, RPT_DEG // 16, abody, None)
    pltpu.sync_copy(acc_v, deg_out.at[c, pl.ds(base, RPT_DEG)])


@functools.partial(
    pl.kernel,
    out_type=jax.ShapeDtypeStruct((NC, N, D), f32),
    mesh=_mesh,
    scratch_types=[
        pltpu.VMEM((EPT,), jnp.int32),
        pltpu.VMEM((EPT,), jnp.int32),
        pltpu.VMEM((K, D), f32),
        pltpu.VMEM((K, D), f32),
        pltpu.VMEM_SHARED((ACC, D), f32),
        pltpu.SemaphoreType.DMA,
        pltpu.SemaphoreType.DMA,
    ],
    compiler_params=_sc_params,
)
def _scatter_kernel(src_hbm, dst_hbm, h_hbm, out_hbm,
                    src_v, dst_v, rows_a, rows_b, acc_sh, sem_a, sem_b):
    c = lax.axis_index("c")
    s = lax.axis_index("s")
    w = c * NS + s
    pltpu.sync_copy(src_hbm.at[w], src_v)
    pltpu.sync_copy(dst_hbm.at[w], dst_v)

    zero16 = jnp.zeros((16,), f32)

    def zrow(i, carry):
        for j in range(D // 16):
            rows_a[i, pl.ds(j * 16, 16)] = zero16
        return carry
    lax.fori_loop(0, K, zrow, None)

    for i in range(-(-(ACC // K) // NS)):
        m = i * NS + s

        @pl.when(m < ACC // K)
        def _():
            pltpu.sync_copy(rows_a, acc_sh.at[pl.ds(m * K, K)])
    plsc.subcore_barrier()

    pltpu.async_copy(h_hbm.at[src_v.at[pl.ds(0, K)]], rows_a, sem_a)

    def pair(j2, carry):
        j = j2 * 2 * K
        pltpu.make_async_copy(
            h_hbm.at[src_v.at[pl.ds(j, K)]], rows_a, sem_a).wait()
        pltpu.async_copy(h_hbm.at[src_v.at[pl.ds(j + K, K)]], rows_b, sem_b)
        pltpu.sync_copy(rows_a, acc_sh.at[dst_v.at[pl.ds(j, K)]], add=True)
        pltpu.make_async_copy(
            h_hbm.at[src_v.at[pl.ds(j + K, K)]], rows_b, sem_b).wait()
        pltpu.async_copy(h_hbm.at[src_v.at[pl.ds(j + 2 * K, K)]], rows_a,
                         sem_a)
        pltpu.sync_copy(rows_b, acc_sh.at[dst_v.at[pl.ds(j + K, K)]],
                        add=True)
        return carry
    lax.fori_loop(0, CH // 2, pair, None)
    pltpu.make_async_copy(
        h_hbm.at[src_v.at[pl.ds((CH - 1) * K, K)]], rows_a, sem_a).wait()
    pltpu.sync_copy(rows_a, acc_sh.at[dst_v.at[pl.ds((CH - 1) * K, K)]],
                    add=True)

    plsc.subcore_barrier()

    for i in range(-(-(ACC // K) // NS)):
        m = i * NS + s

        @pl.when(m < ACC // K)
        def _(m=m):
            pltpu.sync_copy(acc_sh.at[pl.ds(m * K, K)], rows_a)
            pltpu.sync_copy(rows_a, out_hbm.at[c, pl.ds(m * K, K)])


_BLK = 2000
_GRID = N // _BLK


def _mm_scale_body(x_ref, w_ref, b_ref, degs_ref, o_ref):
    h = jnp.dot(x_ref[...], w_ref[...], preferred_element_type=f32) + b_ref[...]
    o_ref[...] = h * lax.rsqrt(jnp.maximum(degs_ref[...], 1.0))


def _tc_mm_scale(x, w, b2d, degs):
    return pl.pallas_call(
        _mm_scale_body,
        grid=(_GRID,),
        in_specs=[
            pl.BlockSpec((_BLK, D), lambda i: (i, 0)),
            pl.BlockSpec((D, D), lambda i: (0, 0)),
            pl.BlockSpec((1, D), lambda i: (0, 0)),
            pl.BlockSpec((_BLK, 1), lambda i: (i, 0)),
        ],
        out_specs=pl.BlockSpec((_BLK, D), lambda i: (i, 0)),
        out_shape=jax.ShapeDtypeStruct((N, D), f32),
    )(x, w, b2d, degs)


def _step_body(p_ref, degd_ref, w_ref, b_ref, degs_ref, xsum_ref,
               xsum_out_ref, h_ref):
    a = (p_ref[0] + p_ref[1]) * lax.rsqrt(jnp.maximum(degd_ref[...], 1.0))
    xn = jnp.where(a >= 0, a, 0.01 * a)
    xsum_out_ref[...] = xsum_ref[...] + xn
    h = jnp.dot(xn, w_ref[...], preferred_element_type=f32) + b_ref[...]
    h_ref[...] = h * lax.rsqrt(jnp.maximum(degs_ref[...], 1.0))


def _tc_step(p, degd, w, b2d, degs, xsum):
    return pl.pallas_call(
        _step_body,
        grid=(_GRID,),
        in_specs=[
            pl.BlockSpec((NC, _BLK, D), lambda i: (0, i, 0)),
            pl.BlockSpec((_BLK, 1), lambda i: (i, 0)),
            pl.BlockSpec((D, D), lambda i: (0, 0)),
            pl.BlockSpec((1, D), lambda i: (0, 0)),
            pl.BlockSpec((_BLK, 1), lambda i: (i, 0)),
            pl.BlockSpec((_BLK, D), lambda i: (i, 0)),
        ],
        out_specs=[
            pl.BlockSpec((_BLK, D), lambda i: (i, 0)),
            pl.BlockSpec((_BLK, D), lambda i: (i, 0)),
        ],
        out_shape=[
            jax.ShapeDtypeStruct((N, D), f32),
            jax.ShapeDtypeStruct((N, D), f32),
        ],
    )(p, degd, w, b2d, degs, xsum)


def _fin_body(x0_ref, xsum_ref, o_ref):
    o_ref[...] = (x0_ref[...] + xsum_ref[...]) * (1.0 / 3.0)


def _tc_fin(x0, xsum):
    return pl.pallas_call(
        _fin_body,
        grid=(_GRID,),
        in_specs=[
            pl.BlockSpec((_BLK, D), lambda i: (i, 0)),
            pl.BlockSpec((_BLK, D), lambda i: (i, 0)),
        ],
        out_specs=pl.BlockSpec((_BLK, D), lambda i: (i, 0)),
        out_shape=jax.ShapeDtypeStruct((N, D), f32),
    )(x0, xsum)


def kernel(edge_index, all_embed, W1, b1, W2, b2):
    ei = edge_index.astype(jnp.int32)
    deg_idx = ei.reshape(NW, EPT2)
    src_r = ei[0].reshape(NW, EPT)
    dst_r = ei[1].reshape(NW, EPT)

    degs = _deg_kernel(deg_idx)
    deg_src = degs[0, :N].reshape(N, 1)
    deg_dst = degs[1, :N].reshape(N, 1)
    Ws = jnp.stack((W1, W2))
    bs = jnp.stack((b1.reshape(1, D), b2.reshape(1, D)))

    nlayers = jnp.int32(2) + jnp.min(ei[0]) * jnp.int32(0)

    def cond(carry):
        i, _, _ = carry
        return i < nlayers

    def body(carry):
        i, h, xsum = carry
        p = _scatter_kernel(src_r, dst_r, h)
        inext = jnp.minimum(i + 1, 1)
        w = lax.dynamic_index_in_dim(Ws, inext, axis=0, keepdims=False)
        b2d = lax.dynamic_index_in_dim(bs, inext, axis=0, keepdims=False)
        xsumn, hn = _tc_step(p, deg_dst, w, b2d, deg_src, xsum)
        return (i + 1, hn, xsumn)

    h0 = _tc_mm_scale(all_embed, W1, bs[0], deg_src)
    z = jnp.zeros((N, D), f32)
    _, _, xsum = lax.while_loop(cond, body, (jnp.int32(0), h0, z))
    return _tc_fin(all_embed, xsum)
